# baseline (speedup 1.0000x reference)
import functools

import jax
import jax.numpy as jnp
from jax import lax
from jax.experimental import pallas as pl
from jax.experimental.pallas import tpu as pltpu
from jax.experimental.pallas import tpu_sc as plsc

EMB_DIM = 64
BATCH = 16384

_NC = 2
_NS = 16
_NW = _NC * _NS
_BPW = BATCH // _NW
_L = 16
_NG = _BPW // _L


def _sc_kernel_body(x_hbm, ptab_hbm, itab_hbm, out_hbm,
                    x_v, idx0_v, idx1_v, rows0_v, rows1_v, out_v,
                    sem0, sem1):
    wid = lax.axis_index("s") * _NC + lax.axis_index("c")
    base = wid * _BPW

    pltpu.sync_copy(x_hbm.at[pl.ds(base, _BPW), :], x_v)

    lane = lax.iota(jnp.int32, _L)
    zero16 = jnp.zeros((_L,), jnp.int32)
    one16 = jnp.ones((_L,), jnp.int32)

    def deint_body(g, _):
        row_ids = g * _L + lane
        idx0_v[pl.ds(g * _L, _L)] = plsc.load_gather(x_v, [row_ids, zero16])
        idx1_v[pl.ds(g * _L, _L)] = plsc.load_gather(x_v, [row_ids, one16])
        return 0

    lax.fori_loop(0, _NG, deint_body, 0, unroll=4)

    cp0 = pltpu.async_copy(ptab_hbm.at[idx0_v], rows0_v, sem0)
    cp1 = pltpu.async_copy(itab_hbm.at[idx1_v], rows1_v, sem1)
    cp0.wait()
    cp1.wait()

    def group_body(g, _):
        row_ids = g * _L + lane
        acc = jnp.zeros((_L,), jnp.float32)

        def d_body(t, acc):
            dcol = jnp.bitwise_and(t + lane, EMB_DIM - 1)
            a = plsc.load_gather(rows0_v, [row_ids, dcol])
            b = plsc.load_gather(rows1_v, [row_ids, dcol])
            return acc + a * b

        acc = lax.fori_loop(0, EMB_DIM, d_body, acc, unroll=8)
        out_v[pl.ds(g * _L, _L)] = acc
        return 0

    lax.fori_loop(0, _NG, group_body, 0)

    pltpu.sync_copy(out_v, out_hbm.at[pl.ds(base, _BPW)])


@jax.jit
def _gmf_dot(x, ptab, itab):
    mesh = plsc.VectorSubcoreMesh(core_axis_name="c", subcore_axis_name="s")
    kern = functools.partial(
        pl.kernel,
        mesh=mesh,
        out_type=jax.ShapeDtypeStruct((BATCH,), jnp.float32),
        scratch_types=[
            pltpu.VMEM((_BPW, 2), jnp.int32),
            pltpu.VMEM((_BPW,), jnp.int32),
            pltpu.VMEM((_BPW,), jnp.int32),
            pltpu.VMEM((_BPW, EMB_DIM), jnp.float32),
            pltpu.VMEM((_BPW, EMB_DIM), jnp.float32),
            pltpu.VMEM((_BPW,), jnp.float32),
            pltpu.SemaphoreType.DMA,
            pltpu.SemaphoreType.DMA,
        ],
        compiler_params=pltpu.CompilerParams(
            use_tc_tiling_on_sc=False, needs_layout_passes=False
        ),
    )(_sc_kernel_body)
    return kern(x, ptab, itab)


def kernel(x, playlist_table, item_table, fc1_w, fc1_b, fc2_w, fc2_b):
    s = jnp.float32(1.0 + 1e-7)
    ptab = playlist_table * s
    itab = item_table * s
    y = _gmf_dot(x.astype(jnp.int32), ptab, itab)
    return y.reshape(BATCH, 1)


# speedup vs baseline: 1.5709x; 1.5709x over previous
import functools

import jax
import jax.numpy as jnp
from jax import lax
from jax.experimental import pallas as pl
from jax.experimental.pallas import tpu as pltpu
from jax.experimental.pallas import tpu_sc as plsc

EMB_DIM = 64
PAD_DIM = 128
BATCH = 16384

_NC = 2
_NS = 16
_NW = _NC * _NS
_BPW = BATCH // _NW
_L = 16
_NG = _BPW // _L


_CH = 4
_CR = _BPW // _CH


def _sc_kernel_body(idx0_hbm, idx1_hbm, ptab_hbm, itab_hbm, out_hbm,
                    idx0_v, idx1_v, r0a, r0b, r1a, r1b, out_v,
                    s0a, s0b, s1a, s1b):
    wid = lax.axis_index("s") * _NC + lax.axis_index("c")
    base = wid * _BPW

    pltpu.sync_copy(idx0_hbm.at[pl.ds(base, _BPW)], idx0_v)
    pltpu.sync_copy(idx1_hbm.at[pl.ds(base, _BPW)], idx1_v)

    bufs0 = (r0a, r0b)
    bufs1 = (r1a, r1b)
    sems0 = (s0a, s0b)
    sems1 = (s1a, s1b)

    def fire(c):
        b = c % 2
        cp0 = pltpu.async_copy(
            ptab_hbm.at[idx0_v.at[pl.ds(c * _CR, _CR)]], bufs0[b], sems0[b])
        cp1 = pltpu.async_copy(
            itab_hbm.at[idx1_v.at[pl.ds(c * _CR, _CR)]], bufs1[b], sems1[b])
        return cp0, cp1

    lane = lax.iota(jnp.int32, _L)
    inflight = {0: fire(0)}

    for c in range(_CH):
        if c + 1 < _CH:
            inflight[c + 1] = fire(c + 1)
        cp0, cp1 = inflight.pop(c)
        cp0.wait()
        cp1.wait()
        b = c % 2
        rows0_v = bufs0[b]
        rows1_v = bufs1[b]

        def group_body(g, _, rows0_v=rows0_v, rows1_v=rows1_v, c=c):
            row_ids = g * _L + lane
            acc = jnp.zeros((_L,), jnp.float32)

            def d_body(t, acc):
                dcol = jnp.bitwise_and(t + lane, EMB_DIM - 1)
                a = plsc.load_gather(rows0_v, [row_ids, dcol])
                b_ = plsc.load_gather(rows1_v, [row_ids, dcol])
                return acc + a * b_

            acc = lax.fori_loop(0, EMB_DIM, d_body, acc, unroll=8)
            out_v[pl.ds(c * _CR + g * _L, _L)] = acc
            return 0

        lax.fori_loop(0, _CR // _L, group_body, 0)

    pltpu.sync_copy(out_v, out_hbm.at[pl.ds(base, _BPW)])


@jax.jit
def _gmf_dot(idx0, idx1, ptab, itab):
    mesh = plsc.VectorSubcoreMesh(core_axis_name="c", subcore_axis_name="s")
    kern = functools.partial(
        pl.kernel,
        mesh=mesh,
        out_type=jax.ShapeDtypeStruct((BATCH,), jnp.float32),
        scratch_types=[
            pltpu.VMEM((_BPW,), jnp.int32),
            pltpu.VMEM((_BPW,), jnp.int32),
            pltpu.VMEM((_CR, PAD_DIM), jnp.float32),
            pltpu.VMEM((_CR, PAD_DIM), jnp.float32),
            pltpu.VMEM((_CR, PAD_DIM), jnp.float32),
            pltpu.VMEM((_CR, PAD_DIM), jnp.float32),
            pltpu.VMEM((_BPW,), jnp.float32),
            pltpu.SemaphoreType.DMA,
            pltpu.SemaphoreType.DMA,
            pltpu.SemaphoreType.DMA,
            pltpu.SemaphoreType.DMA,
        ],
        compiler_params=pltpu.CompilerParams(
            use_tc_tiling_on_sc=False, needs_layout_passes=False
        ),
    )(_sc_kernel_body)
    return kern(idx0, idx1, ptab, itab)


def kernel(x, playlist_table, item_table, fc1_w, fc1_b, fc2_w, fc2_b):
    idx0 = x[:, 0].astype(jnp.int32)
    idx1 = x[:, 1].astype(jnp.int32)
    ptab = jnp.pad(playlist_table, ((0, 0), (0, PAD_DIM - EMB_DIM)))
    itab = jnp.pad(item_table, ((0, 0), (0, PAD_DIM - EMB_DIM)))
    y = _gmf_dot(idx0, idx1, ptab, itab)
    return y.reshape(BATCH, 1)
